# Initial kernel scaffold; baseline (speedup 1.0000x reference)
import jax
import jax.numpy as jnp
from jax.experimental import pallas as pl


def kernel(x, adj, W1, a1_src, a1_dst, W2, a2_src, a2_dst, Wd_rna, Wd_dis):
    raise NotImplementedError("write your pallas kernel here")


# speedup vs baseline: 1.5058x; 1.5058x over previous
import functools

import jax
import jax.numpy as jnp
from jax.experimental import pallas as pl
from jax.experimental.pallas import tpu as pltpu

_N = 2048
_TI = 256
_NSTEPS = _N // _TI
_NEG = -9e15
_HI = jax.lax.Precision.HIGHEST


def _expm1_neg(x):
    t = x * (1.0 + x / 2.0 * (1.0 + x / 3.0 * (1.0 + x / 4.0 *
        (1.0 + x / 5.0 * (1.0 + x / 6.0 * (1.0 + x / 7.0))))))
    return jnp.where(x > -0.25, t, jnp.exp(x) - 1.0)


def _mm_bf16(a, b):
    return jnp.dot(a.astype(jnp.bfloat16), b.astype(jnp.bfloat16),
                   preferred_element_type=jnp.float32)


def _proj_kernel(x_ref, W_ref, asm_ref, adm_ref, h_ref, fs_ref, fdT_ref):
    h = _mm_bf16(x_ref[...], W_ref[...])
    h_ref[...] = h
    fs_ref[...] = jnp.dot(h, asm_ref[...], preferred_element_type=jnp.float32,
                          precision=_HI)
    fdT_ref[...] = jax.lax.dot_general(
        adm_ref[...], h, (((0,), (1,)), ((), ())),
        preferred_element_type=jnp.float32, precision=_HI)


def _attn_rows(i, adj_t, h_ref, fs_ref, fdT_ref, H, O, out_write):
    madj = jnp.where(adj_t > 0.0, 0.0, _NEG).astype(jnp.float32)
    for hh in range(H):
        fs = fs_ref[:, hh:hh + 1]
        fd = fdT_ref[hh:hh + 1, :]
        s = fs + fd
        s = jnp.where(s > 0.0, s, 0.2 * s)
        s = s + madj
        m = jnp.max(s, axis=1, keepdims=True)
        p = jnp.exp(s - m)
        denom = jnp.sum(p, axis=1, keepdims=True)
        acc = _mm_bf16(p / denom, h_ref[:, hh * O:(hh + 1) * O])
        out_write(hh, jnp.where(acc > 0.0, acc, _expm1_neg(acc)))


def _attn1_kernel(adj_ref, h_ref, fs_ref, fdT_ref, o_ref, *, H, O):
    i = pl.program_id(0)

    def write(hh, val):
        o_ref[:, hh * O:(hh + 1) * O] = val

    _attn_rows(i, adj_ref[...], h_ref, fs_ref, fdT_ref, H, O, write)


def _attn2_dec_kernel(adj_ref, h_ref, fs_ref, fdT_ref, wr_ref, wd_ref,
                      ret_ref, o_scr, *, H, O):
    i = pl.program_id(0)

    def write(hh, val):
        o_scr[pl.ds(i * _TI, _TI), hh * O:(hh + 1) * O] = val

    _attn_rows(i, adj_ref[...], h_ref, fs_ref, fdT_ref, H, O, write)

    @pl.when(i == _NSTEPS - 1)
    def _():
        half = _N // 2
        rna = _mm_bf16(o_scr[0:half, :], wr_ref[...])
        dis = _mm_bf16(o_scr[half:_N, :], wd_ref[...])
        ret_ref[...] = jax.lax.dot_general(
            rna.astype(jnp.bfloat16), dis.astype(jnp.bfloat16),
            (((1,), (1,)), ((), ())),
            preferred_element_type=jnp.float32)


def _expand_attn_vec(a):
    Hh, Oo = a.shape
    eye = jnp.eye(Hh, dtype=a.dtype)
    return (a[:, :, None] * eye[:, None, :]).reshape(Hh * Oo, Hh)


def _proj(x, W, asm, adm, H):
    n, d_in = x.shape
    d = W.shape[1]
    return pl.pallas_call(
        _proj_kernel,
        out_shape=(
            jax.ShapeDtypeStruct((n, d), jnp.float32),
            jax.ShapeDtypeStruct((n, H), jnp.float32),
            jax.ShapeDtypeStruct((H, n), jnp.float32),
        ),
    )(x, W, asm, adm)


def kernel(x, adj, W1, a1_src, a1_dst, W2, a2_src, a2_dst, Wd_rna, Wd_dis):
    n = x.shape[0]
    H1, O1 = a1_src.shape
    H2, O2 = a2_src.shape
    d1 = H1 * O1
    d2 = H2 * O2
    half = n // 2

    full = lambda shape: pl.BlockSpec(shape, lambda i: (0, 0))

    h1p, f1s, f1dT = _proj(x, W1, _expand_attn_vec(a1_src),
                           _expand_attn_vec(a1_dst), H1)

    h1 = pl.pallas_call(
        functools.partial(_attn1_kernel, H=H1, O=O1),
        grid=(_NSTEPS,),
        in_specs=[
            pl.BlockSpec((_TI, n), lambda i: (i, 0)),
            full((n, d1)),
            pl.BlockSpec((_TI, H1), lambda i: (i, 0)),
            full((H1, n)),
        ],
        out_specs=pl.BlockSpec((_TI, d1), lambda i: (i, 0)),
        out_shape=jax.ShapeDtypeStruct((n, d1), jnp.float32),
    )(adj, h1p, f1s, f1dT)

    h2p, f2s, f2dT = _proj(h1, W2, _expand_attn_vec(a2_src),
                           _expand_attn_vec(a2_dst), H2)

    ret = pl.pallas_call(
        functools.partial(_attn2_dec_kernel, H=H2, O=O2),
        grid=(_NSTEPS,),
        in_specs=[
            pl.BlockSpec((_TI, n), lambda i: (i, 0)),
            full((n, d2)),
            pl.BlockSpec((_TI, H2), lambda i: (i, 0)),
            full((H2, n)),
            full((d2, Wd_rna.shape[1])),
            full((d2, Wd_dis.shape[1])),
        ],
        out_specs=full((half, half)),
        out_shape=jax.ShapeDtypeStruct((half, half), jnp.float32),
        scratch_shapes=[pltpu.VMEM((n, d2), jnp.float32)],
    )(adj, h2p, f2s, f2dT, Wd_rna, Wd_dis)

    return ret.reshape(-1)


# speedup vs baseline: 1.6353x; 1.0860x over previous
import functools

import jax
import jax.numpy as jnp
from jax.experimental import pallas as pl
from jax.experimental.pallas import tpu as pltpu

_N = 2048
_TI = 256
_NSTEPS = _N // _TI
_NEG = -9e15
_LOG2E = 1.4426950408889634


def _expm1_neg(x):
    t = x * (1.0 + x / 2.0 * (1.0 + x / 3.0 * (1.0 + x / 4.0 *
        (1.0 + x / 5.0 * (1.0 + x / 6.0 * (1.0 + x / 7.0))))))
    return jnp.where(x > -0.25, t, jnp.exp(x) - 1.0)


def _mm_bf16(a, b):
    return jnp.dot(a.astype(jnp.bfloat16), b.astype(jnp.bfloat16),
                   preferred_element_type=jnp.float32)


def _proj_kernel(x_ref, W_ref, asrc_ref, adst_ref, h_ref, fs_ref, fdT_ref,
                 *, H, O):
    h = _mm_bf16(x_ref[...], W_ref[...])
    h_ref[...] = h
    fd_cols = []
    for hh in range(H):
        hs = h[:, hh * O:(hh + 1) * O]
        fs_ref[:, hh:hh + 1] = jnp.sum(hs * asrc_ref[hh:hh + 1, :], axis=1,
                                       keepdims=True)
        fd_cols.append(jnp.sum(hs * adst_ref[hh:hh + 1, :], axis=1,
                               keepdims=True))
    fdT_ref[...] = jnp.concatenate(fd_cols, axis=1).T


def _attn_rows(i, adj_t, h_ref, fs_ref, fdT_ref, H, O, out_write):
    madj = jnp.where(adj_t > 0.0, 0.0, _NEG).astype(jnp.float32)
    for hh in range(H):
        fs = fs_ref[:, hh:hh + 1]
        fd = fdT_ref[hh:hh + 1, :]
        s0 = fs + fd
        s = jnp.maximum(s0, 0.2 * s0) + madj
        m = jnp.max(s, axis=1, keepdims=True)
        p = jnp.exp2(s * _LOG2E - m * _LOG2E)
        denom = jnp.sum(p, axis=1, keepdims=True)
        acc = _mm_bf16(p / denom, h_ref[:, hh * O:(hh + 1) * O])
        out_write(hh, jnp.where(acc > 0.0, acc, _expm1_neg(acc)))


def _attn1_kernel(adj_ref, h_ref, fs_ref, fdT_ref, o_ref, *, H, O):
    i = pl.program_id(0)

    def write(hh, val):
        o_ref[:, hh * O:(hh + 1) * O] = val

    _attn_rows(i, adj_ref[...], h_ref, fs_ref, fdT_ref, H, O, write)


def _attn2_dec_kernel(adj_ref, h_ref, fs_ref, fdT_ref, wr_ref, wd_ref,
                      ret_ref, o_scr, *, H, O):
    i = pl.program_id(0)

    def write(hh, val):
        o_scr[pl.ds(i * _TI, _TI), hh * O:(hh + 1) * O] = val

    _attn_rows(i, adj_ref[...], h_ref, fs_ref, fdT_ref, H, O, write)

    @pl.when(i == _NSTEPS - 1)
    def _():
        half = _N // 2
        rna = _mm_bf16(o_scr[0:half, :], wr_ref[...])
        dis = _mm_bf16(o_scr[half:_N, :], wd_ref[...])
        ret_ref[...] = jax.lax.dot_general(
            rna.astype(jnp.bfloat16), dis.astype(jnp.bfloat16),
            (((1,), (1,)), ((), ())),
            preferred_element_type=jnp.float32)


def _proj(x, W, a_src, a_dst):
    n = x.shape[0]
    d = W.shape[1]
    H = a_src.shape[0]
    return pl.pallas_call(
        functools.partial(_proj_kernel, H=H, O=a_src.shape[1]),
        out_shape=(
            jax.ShapeDtypeStruct((n, d), jnp.float32),
            jax.ShapeDtypeStruct((n, H), jnp.float32),
            jax.ShapeDtypeStruct((H, n), jnp.float32),
        ),
    )(x, W, a_src, a_dst)


def kernel(x, adj, W1, a1_src, a1_dst, W2, a2_src, a2_dst, Wd_rna, Wd_dis):
    n = x.shape[0]
    H1, O1 = a1_src.shape
    H2, O2 = a2_src.shape
    d1 = H1 * O1
    d2 = H2 * O2
    half = n // 2

    full = lambda shape: pl.BlockSpec(shape, lambda i: (0, 0))

    h1p, f1s, f1dT = _proj(x, W1, a1_src, a1_dst)

    h1 = pl.pallas_call(
        functools.partial(_attn1_kernel, H=H1, O=O1),
        grid=(_NSTEPS,),
        in_specs=[
            pl.BlockSpec((_TI, n), lambda i: (i, 0)),
            full((n, d1)),
            pl.BlockSpec((_TI, H1), lambda i: (i, 0)),
            full((H1, n)),
        ],
        out_specs=pl.BlockSpec((_TI, d1), lambda i: (i, 0)),
        out_shape=jax.ShapeDtypeStruct((n, d1), jnp.float32),
    )(adj, h1p, f1s, f1dT)

    h2p, f2s, f2dT = _proj(h1, W2, a2_src, a2_dst)

    ret = pl.pallas_call(
        functools.partial(_attn2_dec_kernel, H=H2, O=O2),
        grid=(_NSTEPS,),
        in_specs=[
            pl.BlockSpec((_TI, n), lambda i: (i, 0)),
            full((n, d2)),
            pl.BlockSpec((_TI, H2), lambda i: (i, 0)),
            full((H2, n)),
            full((d2, Wd_rna.shape[1])),
            full((d2, Wd_dis.shape[1])),
        ],
        out_specs=full((half, half)),
        out_shape=jax.ShapeDtypeStruct((half, half), jnp.float32),
        scratch_shapes=[pltpu.VMEM((n, d2), jnp.float32)],
    )(adj, h2p, f2s, f2dT, Wd_rna, Wd_dis)

    return ret.reshape(-1)


# speedup vs baseline: 1.7643x; 1.0789x over previous
import functools

import jax
import jax.numpy as jnp
from jax.experimental import pallas as pl
from jax.experimental.pallas import tpu as pltpu

_N = 2048
_TI = 256
_NSTEPS = _N // _TI
_NEG = -9e15
_LOG2E = 1.4426950408889634


def _expm1_neg(x):
    t = x * (1.0 + x / 2.0 * (1.0 + x / 3.0 * (1.0 + x / 4.0 *
        (1.0 + x / 5.0 * (1.0 + x / 6.0 * (1.0 + x / 7.0))))))
    return jnp.where(x > -0.25, t, jnp.exp(x) - 1.0)


def _mm_bf16(a, b):
    return jnp.dot(a.astype(jnp.bfloat16), b.astype(jnp.bfloat16),
                   preferred_element_type=jnp.float32)


def _proj_body(xin, W_ref, asrc_ref, adst_ref, h_scr, fs_scr, fdT_scr, H, O):
    h = _mm_bf16(xin, W_ref[...])
    h_scr[...] = h
    fd_cols = []
    for hh in range(H):
        hs = h[:, hh * O:(hh + 1) * O]
        fs_scr[:, hh:hh + 1] = jnp.sum(hs * asrc_ref[hh:hh + 1, :], axis=1,
                                       keepdims=True)
        fd_cols.append(jnp.sum(hs * adst_ref[hh:hh + 1, :], axis=1,
                               keepdims=True))
    fdT_scr[...] = jnp.concatenate(fd_cols, axis=1).T


def _attn_rows(row0, adj_t, h_scr, fs_scr, fdT_scr, H, O, out_write):
    madj = jnp.where(adj_t > 0.0, 0.0, _NEG).astype(jnp.float32)
    for hh in range(H):
        fs = fs_scr[pl.ds(row0, _TI), hh:hh + 1]
        fd = fdT_scr[hh:hh + 1, :]
        s0 = fs + fd
        s = jnp.maximum(s0, 0.2 * s0) + madj
        m = jnp.max(s, axis=1, keepdims=True)
        p = jnp.exp2(s * _LOG2E - m * _LOG2E)
        denom = jnp.sum(p, axis=1, keepdims=True)
        acc = _mm_bf16(p / denom, h_scr[:, hh * O:(hh + 1) * O])
        out_write(hh, jnp.where(acc > 0.0, acc, _expm1_neg(acc)))


def _mega_kernel(x_ref, adj_ref, W1_ref, a1s_ref, a1d_ref,
                 W2_ref, a2s_ref, a2d_ref, wr_ref, wd_ref, ret_ref,
                 h1p, f1s, f1dT, h1, h2p, f2s, f2dT, h2,
                 *, H1, O1, H2, O2):
    i = pl.program_id(0)

    @pl.when(i == 0)
    def _():
        _proj_body(x_ref[...], W1_ref, a1s_ref, a1d_ref, h1p, f1s, f1dT,
                   H1, O1)

    @pl.when((i >= 1) & (i <= _NSTEPS))
    def _():
        row0 = (i - 1) * _TI

        def write(hh, val):
            h1[pl.ds(row0, _TI), hh * O1:(hh + 1) * O1] = val

        _attn_rows(row0, adj_ref[...], h1p, f1s, f1dT, H1, O1, write)

    @pl.when(i == _NSTEPS + 1)
    def _():
        _proj_body(h1[...], W2_ref, a2s_ref, a2d_ref, h2p, f2s, f2dT,
                   H2, O2)

    @pl.when(i >= _NSTEPS + 2)
    def _():
        row0 = (i - _NSTEPS - 2) * _TI

        def write(hh, val):
            h2[pl.ds(row0, _TI), hh * O2:(hh + 1) * O2] = val

        _attn_rows(row0, adj_ref[...], h2p, f2s, f2dT, H2, O2, write)

    @pl.when(i == 2 * _NSTEPS + 1)
    def _():
        half = _N // 2
        rna = _mm_bf16(h2[0:half, :], wr_ref[...])
        dis = _mm_bf16(h2[half:_N, :], wd_ref[...])
        ret_ref[...] = jax.lax.dot_general(
            rna.astype(jnp.bfloat16), dis.astype(jnp.bfloat16),
            (((1,), (1,)), ((), ())),
            preferred_element_type=jnp.float32)


def kernel(x, adj, W1, a1_src, a1_dst, W2, a2_src, a2_dst, Wd_rna, Wd_dis):
    n, d_in = x.shape
    H1, O1 = a1_src.shape
    H2, O2 = a2_src.shape
    d1 = H1 * O1
    d2 = H2 * O2
    half = n // 2

    full = lambda shape: pl.BlockSpec(shape, lambda i: (0, 0))

    def adj_map(i):
        j = jnp.where(i <= _NSTEPS, i - 1, i - _NSTEPS - 2)
        return jnp.clip(j, 0, _NSTEPS - 1), 0

    ret = pl.pallas_call(
        functools.partial(_mega_kernel, H1=H1, O1=O1, H2=H2, O2=O2),
        grid=(2 * _NSTEPS + 2,),
        in_specs=[
            full((n, d_in)),
            pl.BlockSpec((_TI, n), adj_map),
            full((d_in, d1)),
            full((H1, O1)),
            full((H1, O1)),
            full((d1, d2)),
            full((H2, O2)),
            full((H2, O2)),
            full((d2, Wd_rna.shape[1])),
            full((d2, Wd_dis.shape[1])),
        ],
        out_specs=full((half, half)),
        out_shape=jax.ShapeDtypeStruct((half, half), jnp.float32),
        scratch_shapes=[
            pltpu.VMEM((n, d1), jnp.float32),
            pltpu.VMEM((n, H1), jnp.float32),
            pltpu.VMEM((H1, n), jnp.float32),
            pltpu.VMEM((n, d1), jnp.float32),
            pltpu.VMEM((n, d2), jnp.float32),
            pltpu.VMEM((n, H2), jnp.float32),
            pltpu.VMEM((H2, n), jnp.float32),
            pltpu.VMEM((n, d2), jnp.float32),
        ],
    )(x, adj, W1, a1_src, a1_dst, W2, a2_src, a2_dst, Wd_rna, Wd_dis)

    return ret.reshape(-1)


# speedup vs baseline: 1.8553x; 1.0516x over previous
import functools

import jax
import jax.numpy as jnp
from jax.experimental import pallas as pl
from jax.experimental.pallas import tpu as pltpu

_N = 2048
_TI = 256
_NSTEPS = _N // _TI
_LOG2E = 1.4426950408889634
_NEG2 = -9e15 * _LOG2E


def _expm1_neg(x):
    t = x * (1.0 + x / 2.0 * (1.0 + x / 3.0 * (1.0 + x / 4.0 *
        (1.0 + x / 5.0 * (1.0 + x / 6.0 * (1.0 + x / 7.0))))))
    return jnp.where(x > -0.25, t, jnp.exp(x) - 1.0)


def _mm_bf16(a, b):
    return jnp.dot(a.astype(jnp.bfloat16), b.astype(jnp.bfloat16),
                   preferred_element_type=jnp.float32)


def _proj_body(xin, W_ref, asrc_ref, adst_ref, hb_scr, fs_scr,
               fdT_scr, H, O):
    h = _mm_bf16(xin, W_ref[...])
    hb_scr[...] = h.astype(jnp.bfloat16)
    fd_cols = []
    for hh in range(H):
        hs = h[:, hh * O:(hh + 1) * O]
        fs_scr[:, hh:hh + 1] = _LOG2E * jnp.sum(
            hs * asrc_ref[hh:hh + 1, :], axis=1, keepdims=True)
        fd_cols.append(jnp.sum(hs * adst_ref[hh:hh + 1, :], axis=1,
                               keepdims=True))
    fdT_scr[...] = _LOG2E * jnp.concatenate(fd_cols, axis=1).T


def _attn_rows(row0, adj_t, hb_scr, fs_scr, fdT_scr, H, O, out_write):
    madj = jnp.where(adj_t > 0.0, 0.0, _NEG2).astype(jnp.float32)
    for hh in range(H):
        fs = fs_scr[pl.ds(row0, _TI), hh:hh + 1]
        fd = fdT_scr[hh:hh + 1, :]
        s0 = fs + fd
        s = jnp.maximum(s0, 0.2 * s0) + madj
        m = jnp.max(s, axis=1, keepdims=True)
        p = jnp.exp2(s - m)
        denom = jnp.sum(p, axis=1, keepdims=True)
        acc = jnp.dot((p / denom).astype(jnp.bfloat16),
                      hb_scr[:, hh * O:(hh + 1) * O],
                      preferred_element_type=jnp.float32)
        out_write(hh, jnp.where(acc > 0.0, acc, _expm1_neg(acc)))


def _mega_kernel(x_ref, adj_ref, W1_ref, a1s_ref, a1d_ref,
                 W2_ref, a2s_ref, a2d_ref, wr_ref, wd_ref, ret_ref,
                 h1pb, f1s, f1dT, h1, h2pb, f2s, f2dT, h2,
                 *, H1, O1, H2, O2):
    i = pl.program_id(0)

    @pl.when(i == 0)
    def _():
        _proj_body(x_ref[...], W1_ref, a1s_ref, a1d_ref, h1pb,
                   f1s, f1dT, H1, O1)

    @pl.when((i >= 1) & (i <= _NSTEPS))
    def _():
        row0 = (i - 1) * _TI

        def write(hh, val):
            h1[pl.ds(row0, _TI), hh * O1:(hh + 1) * O1] = (
                val.astype(jnp.bfloat16))

        _attn_rows(row0, adj_ref[...], h1pb, f1s, f1dT, H1, O1, write)

    @pl.when(i == _NSTEPS + 1)
    def _():
        _proj_body(h1[...], W2_ref, a2s_ref, a2d_ref,
                   h2pb, f2s, f2dT, H2, O2)

    @pl.when(i >= _NSTEPS + 2)
    def _():
        row0 = (i - _NSTEPS - 2) * _TI

        def write(hh, val):
            h2[pl.ds(row0, _TI), hh * O2:(hh + 1) * O2] = (
                val.astype(jnp.bfloat16))

        _attn_rows(row0, adj_ref[...], h2pb, f2s, f2dT, H2, O2, write)

    @pl.when(i == 2 * _NSTEPS + 1)
    def _():
        half = _N // 2
        rna = jnp.dot(h2[0:half, :], wr_ref[...].astype(jnp.bfloat16),
                      preferred_element_type=jnp.float32)
        dis = jnp.dot(h2[half:_N, :], wd_ref[...].astype(jnp.bfloat16),
                      preferred_element_type=jnp.float32)
        ret_ref[...] = jax.lax.dot_general(
            rna.astype(jnp.bfloat16), dis.astype(jnp.bfloat16),
            (((1,), (1,)), ((), ())),
            preferred_element_type=jnp.float32)


def kernel(x, adj, W1, a1_src, a1_dst, W2, a2_src, a2_dst, Wd_rna, Wd_dis):
    n, d_in = x.shape
    H1, O1 = a1_src.shape
    H2, O2 = a2_src.shape
    d1 = H1 * O1
    d2 = H2 * O2
    half = n // 2

    full = lambda shape: pl.BlockSpec(shape, lambda i: (0, 0))

    def adj_map(i):
        j = jnp.where(i <= _NSTEPS, i - 1, i - _NSTEPS - 2)
        return jnp.clip(j, 0, _NSTEPS - 1), 0

    ret = pl.pallas_call(
        functools.partial(_mega_kernel, H1=H1, O1=O1, H2=H2, O2=O2),
        grid=(2 * _NSTEPS + 2,),
        in_specs=[
            full((n, d_in)),
            pl.BlockSpec((_TI, n), adj_map),
            full((d_in, d1)),
            full((H1, O1)),
            full((H1, O1)),
            full((d1, d2)),
            full((H2, O2)),
            full((H2, O2)),
            full((d2, Wd_rna.shape[1])),
            full((d2, Wd_dis.shape[1])),
        ],
        out_specs=full((half, half)),
        out_shape=jax.ShapeDtypeStruct((half, half), jnp.float32),
        scratch_shapes=[
            pltpu.VMEM((n, d1), jnp.bfloat16),
            pltpu.VMEM((n, H1), jnp.float32),
            pltpu.VMEM((H1, n), jnp.float32),
            pltpu.VMEM((n, d1), jnp.bfloat16),
            pltpu.VMEM((n, d2), jnp.bfloat16),
            pltpu.VMEM((n, H2), jnp.float32),
            pltpu.VMEM((H2, n), jnp.float32),
            pltpu.VMEM((n, d2), jnp.bfloat16),
        ],
    )(x, adj, W1, a1_src, a1_dst, W2, a2_src, a2_dst, Wd_rna, Wd_dis)

    return ret.reshape(-1)
